# baseline (speedup 1.0000x reference)
import jax
import jax.numpy as jnp
from jax.experimental import pallas as pl
from jax.experimental.pallas import tpu as pltpu

_N = 160000
_BLOCK = 3200
_K = 256
_H1 = 512
_H2 = 64


def _mlp_kernel(e_ref, w1_ref, w2_ref, w3_ref, o_ref):
    h = jnp.dot(e_ref[...].astype(jnp.bfloat16), w1_ref[...].astype(jnp.bfloat16),
                preferred_element_type=jnp.float32)
    h = jnp.maximum(h, 0.0)
    h = jnp.dot(h.astype(jnp.bfloat16), w2_ref[...].astype(jnp.bfloat16),
                preferred_element_type=jnp.float32)
    h = jnp.maximum(h, 0.0)
    o_ref[...] = jnp.dot(h, w3_ref[...], preferred_element_type=jnp.float32)


def kernel(E, w1, w2, w3):
    w1p = jnp.pad(w1, ((0, 0), (0, _H1 - w1.shape[1])))
    w2p = jnp.pad(w2, ((0, _H1 - w2.shape[0]), (0, _H2 - w2.shape[1])))
    w3p = jnp.pad(w3, ((0, _H2 - w3.shape[0]), (0, 0)))
    grid = _N // _BLOCK
    out = pl.pallas_call(
        _mlp_kernel,
        grid=(grid,),
        in_specs=[
            pl.BlockSpec((_BLOCK, _K), lambda i: (i, 0)),
            pl.BlockSpec((_K, _H1), lambda i: (0, 0)),
            pl.BlockSpec((_H1, _H2), lambda i: (0, 0)),
            pl.BlockSpec((_H2, 1), lambda i: (0, 0)),
        ],
        out_specs=pl.BlockSpec((_BLOCK, 1), lambda i: (i, 0)),
        out_shape=jax.ShapeDtypeStruct((_N, 1), jnp.float32),
        compiler_params=pltpu.CompilerParams(
            dimension_semantics=("arbitrary",),
        ),
    )(E, w1p, w2p, w3p)
    return out


# speedup vs baseline: 1.0014x; 1.0014x over previous
import jax
import jax.numpy as jnp
from jax.experimental import pallas as pl
from jax.experimental.pallas import tpu as pltpu

_N = 160000
_BLOCK = 3200
_K = 256
_H1 = 512
_H2 = 64


def _mlp_kernel(e_ref, w1_ref, w2_ref, w3_ref, o_ref):
    h = jnp.dot(e_ref[...].astype(jnp.bfloat16), w1_ref[...].astype(jnp.bfloat16),
                preferred_element_type=jnp.float32)
    h = jnp.maximum(h, 0.0)
    h = jnp.dot(h.astype(jnp.bfloat16), w2_ref[...].astype(jnp.bfloat16),
                preferred_element_type=jnp.float32)
    h = jnp.maximum(h, 0.0)
    o_ref[...] = jnp.dot(h, w3_ref[...], preferred_element_type=jnp.float32)


def kernel(E, w1, w2, w3):
    w1p = jnp.pad(w1, ((0, 0), (0, _H1 - w1.shape[1])))
    w2p = jnp.pad(w2, ((0, _H1 - w2.shape[0]), (0, _H2 - w2.shape[1])))
    w3p = jnp.pad(w3, ((0, _H2 - w3.shape[0]), (0, 0)))
    grid = _N // _BLOCK
    out = pl.pallas_call(
        _mlp_kernel,
        grid=(grid,),
        in_specs=[
            pl.BlockSpec((_BLOCK, _K), lambda i: (i, 0)),
            pl.BlockSpec((_K, _H1), lambda i: (0, 0)),
            pl.BlockSpec((_H1, _H2), lambda i: (0, 0)),
            pl.BlockSpec((_H2, 1), lambda i: (0, 0)),
        ],
        out_specs=pl.BlockSpec((_BLOCK, 1), lambda i: (i, 0)),
        out_shape=jax.ShapeDtypeStruct((_N, 1), jnp.float32),
        compiler_params=pltpu.CompilerParams(
            dimension_semantics=("parallel",),
        ),
    )(E, w1p, w2p, w3p)
    return out


# speedup vs baseline: 1.1000x; 1.0985x over previous
import jax
import jax.numpy as jnp
from jax.experimental import pallas as pl
from jax.experimental.pallas import tpu as pltpu

_N = 160000
_BLOCK = 6400
_K = 256
_H1 = 512
_H2 = 64


def _mlp_kernel(e_ref, w1_ref, w2_ref, w3_ref, o_ref):
    h = jnp.dot(e_ref[...].astype(jnp.bfloat16), w1_ref[...].astype(jnp.bfloat16),
                preferred_element_type=jnp.float32)
    h = jnp.maximum(h, 0.0)
    h = jnp.dot(h.astype(jnp.bfloat16), w2_ref[...].astype(jnp.bfloat16),
                preferred_element_type=jnp.float32)
    h = jnp.maximum(h, 0.0)
    o_ref[...] = jnp.dot(h, w3_ref[...], preferred_element_type=jnp.float32)


def kernel(E, w1, w2, w3):
    w1p = jnp.pad(w1, ((0, 0), (0, _H1 - w1.shape[1])))
    w2p = jnp.pad(w2, ((0, _H1 - w2.shape[0]), (0, _H2 - w2.shape[1])))
    w3p = jnp.pad(w3, ((0, _H2 - w3.shape[0]), (0, 0)))
    grid = _N // _BLOCK
    out = pl.pallas_call(
        _mlp_kernel,
        grid=(grid,),
        in_specs=[
            pl.BlockSpec((_BLOCK, _K), lambda i: (i, 0)),
            pl.BlockSpec((_K, _H1), lambda i: (0, 0)),
            pl.BlockSpec((_H1, _H2), lambda i: (0, 0)),
            pl.BlockSpec((_H2, 1), lambda i: (0, 0)),
        ],
        out_specs=pl.BlockSpec((_BLOCK, 1), lambda i: (i, 0)),
        out_shape=jax.ShapeDtypeStruct((_N, 1), jnp.float32),
        compiler_params=pltpu.CompilerParams(
            dimension_semantics=("parallel",),
        ),
    )(E, w1p, w2p, w3p)
    return out
